# baseline (speedup 1.0000x reference)
import functools

import jax
import jax.numpy as jnp
from jax import lax
from jax.experimental import pallas as pl
from jax.experimental.pallas import tpu as pltpu
from jax.experimental.pallas import tpu_sc as plsc

VOCAB = 1000
EMBED_DIM = 8
B, L = 1024, 20
KP = 128

LH = L // 2
TH = B * LH

NC, NS = 2, 16
NW = NC * NS
ROW_PER_W = TH // NW
CHUNK = 64
NCHUNK = ROW_PER_W // CHUNK

LB = 2


def _gather_body(tab_hbm, idx_hbm, emb_hbm, idx_v, buf0, buf1, g0, g1, s0, s1):
    wid = lax.axis_index("s") * NC + lax.axis_index("c")
    base = wid * ROW_PER_W
    pltpu.sync_copy(idx_hbm.at[pl.ds(base, ROW_PER_W)], idx_v)
    bufs, gsems, ssems = (buf0, buf1), (g0, g1), (s0, s1)

    def fire_gather(c):
        return pltpu.async_copy(
            tab_hbm.at[idx_v.at[pl.ds(c * CHUNK, CHUNK)]],
            bufs[c % 2], gsems[c % 2])

    gathers = [None] * NCHUNK
    stores = [None] * NCHUNK
    gathers[0] = fire_gather(0)
    for c in range(NCHUNK):
        gathers[c].wait()
        if c + 1 < NCHUNK:
            if c >= 1:
                stores[c - 1].wait()
            gathers[c + 1] = fire_gather(c + 1)
        stores[c] = pltpu.async_copy(
            bufs[c % 2], emb_hbm.at[pl.ds(base + c * CHUNK, CHUNK)],
            ssems[c % 2])
    stores[NCHUNK - 2].wait()
    stores[NCHUNK - 1].wait()


_gather_rows = functools.partial(
    pl.kernel,
    out_type=jax.ShapeDtypeStruct((TH, KP), jnp.float32),
    mesh=plsc.VectorSubcoreMesh(core_axis_name="c", subcore_axis_name="s"),
    scratch_types=[
        pltpu.VMEM((ROW_PER_W,), jnp.int32),
        pltpu.VMEM((CHUNK, KP), jnp.float32),
        pltpu.VMEM((CHUNK, KP), jnp.float32),
        pltpu.SemaphoreType.DMA,
        pltpu.SemaphoreType.DMA,
        pltpu.SemaphoreType.DMA,
        pltpu.SemaphoreType.DMA,
    ],
    compiler_params=pltpu.CompilerParams(use_tc_tiling_on_sc=False),
)(_gather_body)


def _proj_body_first(w_ref, emb_ref, out_ref):
    for j in range(LB):
        rhs = emb_ref[j].astype(jnp.bfloat16)
        out_ref[j] = lax.dot_general(
            w_ref[...], rhs,
            dimension_numbers=(((1,), (1,)), ((), ())),
            preferred_element_type=jnp.float32,
        )


def _proj_body_second(w_ref, emb_ref, acc_ref, out_ref):
    del acc_ref
    _proj_body_first(w_ref, emb_ref, out_ref)


def _project_first(w128, emb3):
    return pl.pallas_call(
        _proj_body_first,
        grid=(LH // LB,),
        in_specs=[
            pl.BlockSpec((VOCAB, KP), lambda l: (0, 0)),
            pl.BlockSpec((LB, B, KP), lambda l: (l, 0, 0)),
        ],
        out_specs=pl.BlockSpec((LB, VOCAB, B), lambda l: (l, 0, 0)),
        out_shape=jax.ShapeDtypeStruct((L, VOCAB, B), jnp.float32),
    )(w128, emb3)


def _project_second(w128, emb3, acc):
    return pl.pallas_call(
        _proj_body_second,
        grid=(LH // LB,),
        in_specs=[
            pl.BlockSpec((VOCAB, KP), lambda l: (0, 0)),
            pl.BlockSpec((LB, B, KP), lambda l: (l, 0, 0)),
            pl.BlockSpec(memory_space=pl.ANY),
        ],
        out_specs=pl.BlockSpec((LB, VOCAB, B),
                               lambda l: ((LH // LB) + l, 0, 0)),
        out_shape=jax.ShapeDtypeStruct((L, VOCAB, B), jnp.float32),
        input_output_aliases={2: 0},
    )(w128, emb3, acc)


def kernel(x, embed_table, W, b):
    f32 = jnp.float32
    table128 = jnp.concatenate(
        [embed_table.astype(f32),
         jnp.ones((VOCAB, 1), f32),
         jnp.zeros((VOCAB, KP - EMBED_DIM - 1), f32)], axis=1)
    w128 = jnp.concatenate(
        [W.astype(f32), b.astype(f32)[:, None],
         jnp.zeros((VOCAB, KP - EMBED_DIM - 1), f32)],
        axis=1).astype(jnp.bfloat16)
    idx = x.T.reshape(L * B).astype(jnp.int32)
    emb_a = _gather_rows(table128, idx[:TH])
    emb_b = _gather_rows(table128, idx[TH:])
    acc = _project_first(w128, emb_a.reshape(LH, B, KP))
    acc = _project_second(w128, emb_b.reshape(LH, B, KP), acc)
    return jnp.transpose(acc, (2, 0, 1))
